# baseline (speedup 1.0000x reference)
import functools
import math

import jax
import jax.numpy as jnp
from jax.experimental import pallas as pl
from jax.experimental.pallas import tpu as pltpu
from jax.experimental.pallas import tpu_sc as plsc

NC = 2
NS = 16
NW = NC * NS
LANES = 16
CHUNK = 128


@functools.lru_cache(maxsize=None)
def _build(B, V, D, scale):
    rows_per_w = B // NW
    n_chunks = rows_per_w // CHUNK
    mesh = plsc.VectorSubcoreMesh(
        core_axis_name="c", subcore_axis_name="s",
        num_cores=NC, num_subcores=NS)

    @functools.partial(
        pl.kernel,
        out_type=jax.ShapeDtypeStruct((B, D), jnp.float32),
        mesh=mesh,
        scratch_types=[
            pltpu.VMEM((n_chunks, CHUNK), jnp.int32),
            pltpu.VMEM((CHUNK, D), jnp.float32),
            pltpu.SemaphoreType.DMA,
        ],
        compiler_params=pltpu.CompilerParams(use_tc_tiling_on_sc=False),
    )
    def emb_kernel(idx_hbm, table_hbm, out_hbm, idx_v, rows_v, gsem):
        wid = jax.lax.axis_index("s") * NC + jax.lax.axis_index("c")
        chunk_base = wid * n_chunks
        row_base = wid * rows_per_w
        pltpu.sync_copy(idx_hbm.at[pl.ds(chunk_base, n_chunks)], idx_v)

        @pl.loop(0, n_chunks)
        def _chunk(j):
            pltpu.async_copy(table_hbm.at[idx_v.at[j]], rows_v, gsem).wait()

            @pl.loop(0, CHUNK)
            def _row(r):
                for c in range(D // LANES):
                    sl = pl.ds(c * LANES, LANES)
                    rows_v[r, sl] = rows_v[r, sl] * scale

            pltpu.sync_copy(
                rows_v, out_hbm.at[pl.ds(row_base + j * CHUNK, CHUNK)])

    return emb_kernel


def kernel(x, table):
    V, D = table.shape
    B = x.size
    scale = math.sqrt(D)
    idx = x.reshape(B // CHUNK, CHUNK).astype(jnp.int32)
    out = _build(B, V, D, scale)(idx, table)
    return out.reshape(x.shape + (D,))
